# Initial kernel scaffold; baseline (speedup 1.0000x reference)
import jax
import jax.numpy as jnp
from jax.experimental import pallas as pl


def kernel(x, adj_indices, adj_values, W1, b1, W2, b2, prop):
    raise NotImplementedError("write your pallas kernel here")


# speedup vs baseline: 3.1097x; 3.1097x over previous
import functools

import jax
import jax.numpy as jnp
from jax import lax
from jax.experimental import pallas as pl
from jax.experimental.pallas import tpu as pltpu
from jax.experimental.pallas import tpu_sc as plsc

NN = 10000
NFEAT_ = 128
NH = 64
D = 128
NCLS = 16
EE = 160000
KC = 8
THR_ = 0.5
LAMBD_ = 1.0 / (KC + 1.0 / (THR_ - 0.3))

NC = 2
NS = 16
NW = NC * NS
CB = 128
NP = 10240

f32 = jnp.float32


def _mlp_body(x_ref, w_ref, b_ref, o_ref):
    h = jnp.dot(x_ref[...], w_ref[...], preferred_element_type=f32)
    o_ref[...] = jnp.maximum(h + b_ref[...], 0.0)


def _mlp(x, w80, b80):
    bm = 400
    return pl.pallas_call(
        _mlp_body,
        grid=(NN // bm,),
        in_specs=[
            pl.BlockSpec((bm, NFEAT_), lambda i: (i, 0)),
            pl.BlockSpec((NFEAT_, D), lambda i: (0, 0)),
            pl.BlockSpec((1, D), lambda i: (0, 0)),
        ],
        out_specs=pl.BlockSpec((bm, D), lambda i: (i, 0)),
        out_shape=jax.ShapeDtypeStruct((NN, D), f32),
    )(x, w80, b80)


def _merge_fix_body(p_ref, o_ref):
    t = p_ref[0] + p_ref[1]
    li = lax.broadcasted_iota(jnp.int32, t.shape, 1)
    o_ref[...] = jnp.where(li < NH, t, jnp.where(li == NH, 1.0, 0.0))


def _merge_norm_body(p_ref, o_ref):
    t = p_ref[0] + p_ref[1]
    li = lax.broadcasted_iota(jnp.int32, t.shape, 1)
    tm = jnp.where(li < NH, t, 0.0)
    nrm = jnp.sqrt(jnp.sum(tm * tm, axis=1, keepdims=True))
    o_ref[...] = tm / jnp.maximum(nrm, 1e-12)


def _merge_div_body(p_ref, o_ref):
    t = p_ref[0] + p_ref[1]
    li = lax.broadcasted_iota(jnp.int32, t.shape, 1)
    s = jnp.sum(jnp.where(li == NH, t, 0.0), axis=1, keepdims=True)
    sa = jnp.where(s > 0.0, s, 1.0)
    cur = jnp.where(s > 0.0, t / sa, 0.0)
    o_ref[...] = jnp.where(li < NH, cur, jnp.where(li == NH, 1.0, 0.0))


def _merge(body, parts):
    bm = 400
    return pl.pallas_call(
        body,
        grid=(NN // bm,),
        in_specs=[pl.BlockSpec((2, bm, D), lambda i: (0, i, 0))],
        out_specs=pl.BlockSpec((bm, D), lambda i: (i, 0)),
        out_shape=jax.ShapeDtypeStruct((NN, D), f32),
    )(parts)


BMT = 80


def _cos_topk_body(xnb_ref, xna_ref, inds_ref, keys_ref, ctb_ref):
    i0 = pl.program_id(0) * BMT
    cos = lax.dot_general(
        xnb_ref[...], xna_ref[...], (((1,), (1,)), ((), ())),
        preferred_element_type=f32)
    ci = lax.broadcasted_iota(jnp.int32, (BMT, NN), 1)
    ri = lax.broadcasted_iota(jnp.int32, (BMT, NN), 0) + i0
    work = cos - jnp.where(ci == ri, 1.0, 0.0)
    vcols, icols = [], []
    for k in range(KC):
        m = jnp.max(work, axis=1, keepdims=True)
        am = jnp.min(jnp.where(work == m, ci, NN), axis=1, keepdims=True)
        vcols.append(m)
        icols.append(am)
        if k < KC - 1:
            work = jnp.where(ci == am, -jnp.inf, work)
    vals = jnp.concatenate(vcols, axis=1)
    inds = jnp.concatenate(icols, axis=1)
    inds_ref[...] = inds
    rix = lax.broadcasted_iota(jnp.int32, (BMT, KC), 0) + i0
    kix = lax.broadcasted_iota(jnp.int32, (BMT, KC), 1)
    nz = vals != 0.0
    keys_ref[...] = jnp.where(nz, rix * NN + inds, NN * NN + rix * KC + kix)
    ctb_ref[...] = jnp.where(nz, LAMBD_ * vals, -1e9)


def _cos_topk(xn80):
    return pl.pallas_call(
        _cos_topk_body,
        grid=(NN // BMT,),
        in_specs=[
            pl.BlockSpec((BMT, D), lambda i: (i, 0)),
            pl.BlockSpec((NN, D), lambda i: (0, 0)),
        ],
        out_specs=[
            pl.BlockSpec((BMT, KC), lambda i: (i, 0)),
            pl.BlockSpec((BMT, KC), lambda i: (i, 0)),
            pl.BlockSpec((BMT, KC), lambda i: (i, 0)),
        ],
        out_shape=[
            jax.ShapeDtypeStruct((NN, KC), jnp.int32),
            jax.ShapeDtypeStruct((NN, KC), jnp.int32),
            jax.ShapeDtypeStruct((NN, KC), f32),
        ],
    )(xn80, xn80)


BE = 2048


def _edge_w_body(xr_ref, xc_ref, av_ref, w_ref):
    v = jnp.sum(xr_ref[...] * xc_ref[...], axis=1).reshape(1, 1, BE)
    w_ref[...] = jnp.where(v < THR_, -1e9, av_ref[...])


def _edge_w(xr, xc, av3, e2):
    return pl.pallas_call(
        _edge_w_body,
        grid=(e2 // BE,),
        in_specs=[
            pl.BlockSpec((BE, D), lambda i: (i, 0)),
            pl.BlockSpec((BE, D), lambda i: (i, 0)),
            pl.BlockSpec((1, 1, BE), lambda i: (i, 0, 0)),
        ],
        out_specs=pl.BlockSpec((1, 1, BE), lambda i: (i, 0, 0)),
        out_shape=jax.ShapeDtypeStruct((e2 // BE, 1, BE), f32),
    )(xr, xc, av3)


def _final_body(h_ref, c2_ref, c3_ref, pr_ref, w2_ref, b2_ref, o_ref):
    h = h_ref[...]
    c2 = c2_ref[...]
    c3 = c3_ref[...]
    pr = pr_ref[...]
    k0 = jnp.sum(h * pr, axis=1, keepdims=True)
    k1 = jnp.sum(c2 * pr, axis=1, keepdims=True)
    k2 = jnp.sum(c3 * pr, axis=1, keepdims=True)
    m = jnp.maximum(k0, jnp.maximum(k1, k2))
    e0 = jnp.exp(k0 - m)
    e1 = jnp.exp(k1 - m)
    e2 = jnp.exp(k2 - m)
    den = e0 + e1 + e2
    res = (h * e0 + c2 * e1 + c3 * e2) / den
    o_ref[...] = jnp.dot(res, w2_ref[...], preferred_element_type=f32) + b2_ref[...]


def _final(h80, c2a, c3a, prop80, w2p, b2):
    bm = 400
    return pl.pallas_call(
        _final_body,
        grid=(NN // bm,),
        in_specs=[
            pl.BlockSpec((bm, D), lambda i: (i, 0)),
            pl.BlockSpec((bm, D), lambda i: (i, 0)),
            pl.BlockSpec((bm, D), lambda i: (i, 0)),
            pl.BlockSpec((bm, D), lambda i: (i, 0)),
            pl.BlockSpec((D, NCLS), lambda i: (0, 0)),
            pl.BlockSpec((1, NCLS), lambda i: (0, 0)),
        ],
        out_specs=pl.BlockSpec((bm, NCLS), lambda i: (i, 0)),
        out_shape=jax.ShapeDtypeStruct((NN, NCLS), f32),
    )(h80, c2a, c3a, prop80, w2p, b2.reshape(1, NCLS))


MTS = NW * CB * 60


def _segscan_body(k_ref, v_ref, w_ref):
    k = k_ref[...]
    v = v_ref[...]
    d = 1
    while d < MTS:
        kp = jnp.concatenate(
            [jnp.full((1, d), -1, jnp.int32), k[:, : MTS - d]], axis=1)
        vp = jnp.concatenate(
            [jnp.zeros((1, d), f32), v[:, : MTS - d]], axis=1)
        v = v + jnp.where(k == kp, vp, 0.0)
        d *= 2
    kn = jnp.concatenate([k[:, 1:], jnp.full((1, 1), -1, jnp.int32)], axis=1)
    w_ref[...] = jnp.where(k != kn, jnp.exp(v), 0.0)


def _segscan(sk, sv):
    return pl.pallas_call(
        _segscan_body,
        grid=(1,),
        in_specs=[
            pl.BlockSpec((1, MTS), lambda i: (0, 0)),
            pl.BlockSpec((1, MTS), lambda i: (0, 0)),
        ],
        out_specs=pl.BlockSpec((1, MTS), lambda i: (0, 0)),
        out_shape=jax.ShapeDtypeStruct((1, MTS), f32),
    )(sk, sv)


@functools.lru_cache(maxsize=None)
def _seg_kernel(m_total):
    nchunk = m_total // (NW * CB)
    per_w = nchunk * CB
    stripe = NP // NS
    mesh = plsc.VectorSubcoreMesh(core_axis_name="c", subcore_axis_name="s")

    @functools.partial(
        pl.kernel,
        mesh=mesh,
        out_type=jax.ShapeDtypeStruct((NC, NP, D), f32),
        scratch_types=[
            pltpu.VMEM_SHARED((NP, D), f32),
            pltpu.VMEM((CB,), jnp.int32),
            pltpu.VMEM((CB,), jnp.int32),
            pltpu.VMEM((CB, 16), f32),
            pltpu.VMEM((CB, D), f32),
            pltpu.SemaphoreType.DMA,
        ],
    )
    def k(table, ridx, cidx, wref, zeros, out, acc, rv, cv, wv, rows, sem):
        cid = lax.axis_index("c")
        sid = lax.axis_index("s")
        wid = sid * NC + cid
        pltpu.sync_copy(zeros.at[pl.ds(sid * stripe, stripe)],
                        acc.at[pl.ds(sid * stripe, stripe)])
        plsc.subcore_barrier()

        def chunk(g, carry):
            start = pl.multiple_of(wid * per_w + g * CB, CB)
            pltpu.sync_copy(ridx.at[pl.ds(start, CB)], rv)
            pltpu.sync_copy(cidx.at[pl.ds(start, CB)], cv)
            pltpu.sync_copy(wref.at[pl.ds(start, CB)], wv)
            pltpu.async_copy(table.at[cv], rows, sem).wait()
            for b in range(CB):
                wb = wv[b]
                for f in range(D // 16):
                    rows[b, pl.ds(f * 16, 16)] = rows[b, pl.ds(f * 16, 16)] * wb
            pltpu.sync_copy(rows, acc.at[rv], add=True)
            return carry

        lax.fori_loop(0, nchunk, chunk, 0)
        plsc.subcore_barrier()
        pltpu.sync_copy(acc.at[pl.ds(sid * stripe, stripe)],
                        out.at[cid, pl.ds(sid * stripe, stripe)])

    return k


@functools.lru_cache(maxsize=None)
def _gather2_kernel(m_total):
    nchunk = m_total // (NW * CB)
    per_w = nchunk * CB
    mesh = plsc.VectorSubcoreMesh(core_axis_name="c", subcore_axis_name="s")

    @functools.partial(
        pl.kernel,
        mesh=mesh,
        out_type=[
            jax.ShapeDtypeStruct((m_total, D), f32),
            jax.ShapeDtypeStruct((m_total, D), f32),
        ],
        scratch_types=[
            pltpu.VMEM((CB,), jnp.int32),
            pltpu.VMEM((CB, D), f32),
            pltpu.SemaphoreType.DMA,
        ],
    )
    def k(table, ridx, cidx, outr, outc, iv, rows, sem):
        cid = lax.axis_index("c")
        sid = lax.axis_index("s")
        wid = sid * NC + cid

        def chunk(g, carry):
            start = pl.multiple_of(wid * per_w + g * CB, CB)
            pltpu.sync_copy(ridx.at[pl.ds(start, CB)], iv)
            pltpu.async_copy(table.at[iv], rows, sem).wait()
            pltpu.sync_copy(rows, outr.at[pl.ds(start, CB)])
            pltpu.sync_copy(cidx.at[pl.ds(start, CB)], iv)
            pltpu.async_copy(table.at[iv], rows, sem).wait()
            pltpu.sync_copy(rows, outc.at[pl.ds(start, CB)])
            return carry

        lax.fori_loop(0, nchunk, chunk, 0)

    return k


def _pad_to(a, m):
    return jnp.pad(a, [(0, m - a.shape[0])] + [(0, 0)] * (a.ndim - 1))


def _forward_impl(x, adj_indices, adj_values, W1, b1, W2, b2, prop):
    row = adj_indices[0]
    col = adj_indices[1]
    e2 = NW * CB * 40
    mt = MTS

    w1p = jnp.pad(W1, ((0, 0), (0, D - NH)))
    b1p = jnp.concatenate([b1, jnp.ones((1,), f32), jnp.zeros((D - NH - 1,), f32)])
    w2p = jnp.pad(W2, ((0, D - NH), (0, 0)))
    prop80 = jnp.pad(prop, ((0, 0), (0, D - NH)))
    zeros = jnp.zeros((NP, D), f32)

    rowp = _pad_to(row, e2)
    colp = _pad_to(col, e2)
    valp = _pad_to(adj_values, e2)
    valp16 = jnp.broadcast_to(valp[:, None], (e2, 16))

    h80 = _mlp(x, w1p, b1p.reshape(1, D))

    seg = _seg_kernel(e2)
    agg1 = _merge_fix(seg(h80, rowp, colp, valp16, zeros))
    xn80 = _merge_norm_(seg(agg1, rowp, colp, valp16, zeros))

    inds, tkeys, tctb = _cos_topk(xn80)

    xr, xc = _gather2_kernel(e2)(xn80, rowp, colp)
    w3 = _edge_w(xr, xc, valp.reshape(e2 // BE, 1, BE), e2)
    ce = w3.reshape(-1)[:EE]

    npad = mt - EE - KC * NN
    rows_all = jnp.concatenate(
        [row, jnp.repeat(jnp.arange(NN, dtype=jnp.int32), KC),
         jnp.zeros((npad,), jnp.int32)])
    cols_all = jnp.concatenate(
        [col, inds.reshape(-1), jnp.zeros((npad,), jnp.int32)])
    keys_all = jnp.concatenate(
        [row * NN + col, tkeys.reshape(-1),
         NN * NN + KC * NN + jnp.arange(npad, dtype=jnp.int32)])
    ctb_all = jnp.concatenate(
        [ce, tctb.reshape(-1), jnp.full((npad,), -1e9, f32)])
    order = jnp.argsort(keys_all)
    sr = rows_all[order]
    sc_ = cols_all[order]
    tw = _segscan(keys_all[order].reshape(1, mt),
                  ctb_all[order].reshape(1, mt)).reshape(-1)
    tw16 = jnp.broadcast_to(tw[:, None], (mt, 16))

    seg2 = _seg_kernel(mt)
    c2a = _merge_div_(seg2(h80, sr, sc_, tw16, zeros))
    c3a = _merge_div_(seg2(c2a, sr, sc_, tw16, zeros))

    return _final(h80, c2a, c3a, prop80, w2p, b2)


def _merge_fix(parts):
    return _merge(_merge_fix_body, parts)


def _merge_norm_(parts):
    return _merge(_merge_norm_body, parts)


def _merge_div_(parts):
    return _merge(_merge_div_body, parts)


_forward_jitted = jax.jit(_forward_impl)


def kernel(x, adj_indices, adj_values, W1, b1, W2, b2, prop):
    return _forward_jitted(x, adj_indices, adj_values, W1, b1, W2, b2, prop)
